# Initial kernel scaffold; baseline (speedup 1.0000x reference)
import jax
import jax.numpy as jnp
from jax.experimental import pallas as pl


def kernel(triples, node_embeddings, weights1, bases1, weights2, bases2, bias1, bias2):
    raise NotImplementedError("write your pallas kernel here")


# speedup vs baseline: 42.1085x; 42.1085x over previous
import functools

import jax
import jax.numpy as jnp
from jax import lax
from jax.experimental import pallas as pl
from jax.experimental.pallas import tpu as pltpu
from jax.experimental.pallas import tpu_sc as plsc

N = 10000
NREL = 8
R = 2 * NREL + 1
D = 128
H = 16
C = 50
NB = 16

E = 320000
BINS = 16 * N
HALF = 8 * N

NCORES = 2
NSUB = 16
GRP = 2048
CH = GRP // 128
GROUPS = 10
EPT = GRP * GROUPS
EPC = EPT * NSUB
E2P = EPC * NCORES
PADC = EPC - E
RPT = HALF // NSUB
CSPW = 16 * 5120

_B1 = 2000
_B2 = 1000
_B3 = 1000


def _wts_body(w1_ref, b1_ref, w2_ref, b2_ref, o1_ref, o2_ref):
    o1_ref[...] = jnp.dot(w1_ref[...], b1_ref[...],
                          preferred_element_type=jnp.float32)
    o2_ref[...] = jnp.dot(w2_ref[...], b2_ref[...],
                          preferred_element_type=jnp.float32)


def _h_body(emb_ref, w_ref, o_ref):
    o_ref[...] = jnp.dot(emb_ref[...], w_ref[...],
                         preferred_element_type=jnp.float32)


def _l1_body(g_ref, c_ref, hs_ref, b1_ref, o_ref):
    acc = hs_ref[...] + b1_ref[...]
    for r in range(16):
        cr = c_ref[r]
        cinv = jnp.where(cr > 0.0, 1.0 / cr, 0.0)
        acc = acc + g_ref[r] * cinv
    o_ref[...] = jnp.maximum(acc, 0.0)


def _l2_body(u_ref, c_ref, h1_ref, w2_ref, b2_ref, o_ref):
    parts = []
    for r in range(16):
        cr = c_ref[r]
        cinv = jnp.where(cr > 0.0, 1.0 / cr, 0.0)
        parts.append(u_ref[r] * cinv)
    parts.append(h1_ref[...])
    v = jnp.concatenate(parts, axis=1)
    o_ref[...] = (jnp.dot(v, w2_ref[...], preferred_element_type=jnp.float32)
                  + b2_ref[...])


@functools.lru_cache(maxsize=None)
def _make_edge_scatter(table_rows: int, with_counts: bool):
    mesh = plsc.VectorSubcoreMesh(core_axis_name="c", subcore_axis_name="s")

    out_type = [jax.ShapeDtypeStruct((BINS, H), jnp.float32)]
    if with_counts:
        out_type.append(jax.ShapeDtypeStruct((NCORES * CSPW,), jnp.float32))

    scratch = [
        pltpu.VMEM((CH, 128), jnp.int32),
        pltpu.VMEM((CH, 128), jnp.int32),
        pltpu.VMEM((GRP, H), jnp.float32),
        pltpu.VMEM_SHARED((HALF + 8, H), jnp.float32),
        pltpu.SemaphoreType.DMA,
        pltpu.SemaphoreType.DMA,
    ]
    if with_counts:
        scratch += [
            pltpu.VMEM((128,), jnp.float32),
            pltpu.VMEM((1024,), jnp.float32),
            pltpu.VMEM_SHARED((CSPW,), jnp.float32),
        ]

    @functools.partial(
        pl.kernel, out_type=out_type, mesh=mesh, scratch_types=scratch,
        compiler_params=pltpu.CompilerParams(use_tc_tiling_on_sc=False))
    def sc_kernel(idx_hbm, ver_hbm, table_hbm, *rest):
        if with_counts:
            g_hbm, cnt_hbm = rest[0], rest[1]
            idxv, verv, rows, gsp, gsem, ssem, ones, zbuf, csp = rest[2:]
        else:
            g_hbm = rest[0]
            idxv, verv, rows, gsp, gsem, ssem = rest[1:]
        c = lax.axis_index("c")
        t = lax.axis_index("s")
        trpt = pl.multiple_of(t * RPT, 8)

        def zrow(i, carry):
            rows[i, :] = jnp.zeros((16,), jnp.float32)
            return carry
        lax.fori_loop(0, 1024, zrow, 0)
        for k in range(5):
            sz = 1024 if k < 4 else RPT - 4 * 1024
            pltpu.sync_copy(rows.at[pl.ds(0, sz)],
                            gsp.at[pl.ds(trpt + k * 1024, sz)])

        @pl.when(t == 0)
        def _zero_pad():
            pltpu.sync_copy(rows.at[pl.ds(0, 8)], gsp.at[pl.ds(HALF, 8)])

        if with_counts:
            for j in range(8):
                ones[pl.ds(j * 16, 16)] = jnp.full((16,), 1.0, jnp.float32)

            def zb(i, carry):
                zbuf[pl.ds(i * 16, 16)] = jnp.zeros((16,), jnp.float32)
                return carry
            lax.fori_loop(0, 64, zb, 0)
            tcw = pl.multiple_of(t * 5120, 128)
            for k in range(5):
                pltpu.sync_copy(zbuf, csp.at[pl.ds(tcw + k * 1024, 1024)])

        plsc.subcore_barrier()

        row0 = (c * EPC + t * EPT) // 128

        def group(g, carry):
            base = pl.multiple_of(row0 + g * CH, 8)
            pltpu.sync_copy(idx_hbm.at[pl.ds(base, CH)], idxv)
            pltpu.sync_copy(ver_hbm.at[pl.ds(base, CH)], verv)
            gathers = []
            for j in range(CH):
                gathers.append(pltpu.async_copy(
                    table_hbm.at[idxv.at[j]],
                    rows.at[pl.ds(j * 128, 128)], gsem))
            for cp in gathers:
                cp.wait()
            scatters = []
            for j in range(CH):
                scatters.append(pltpu.async_copy(
                    rows.at[pl.ds(j * 128, 128)],
                    gsp.at[verv.at[j]], ssem, add=True))
                if with_counts:
                    scatters.append(pltpu.async_copy(
                        ones, csp.at[verv.at[j]], ssem, add=True))
            for cp in scatters:
                cp.wait()
            return carry

        lax.fori_loop(0, GROUPS, group, 0)

        plsc.subcore_barrier()

        ob = pl.multiple_of(c * HALF + trpt, 8)
        for k in range(3):
            sz = 2048 if k < 2 else RPT - 2 * 2048
            pltpu.sync_copy(gsp.at[pl.ds(trpt + k * 2048, sz)],
                            rows.at[pl.ds(0, sz)])
            pltpu.sync_copy(rows.at[pl.ds(0, sz)],
                            g_hbm.at[pl.ds(ob + k * 2048, sz)])
        if with_counts:
            tcw = pl.multiple_of(t * 5120, 128)
            cb = pl.multiple_of(c * CSPW, 128)
            for k in range(5):
                pltpu.sync_copy(csp.at[pl.ds(tcw + k * 1024, 1024)], zbuf)
                pltpu.sync_copy(zbuf,
                                cnt_hbm.at[pl.ds(cb + tcw + k * 1024, 1024)])

    return sc_kernel


def _pad_to_rows(x, fill):
    half = jnp.full((PADC,), fill, jnp.int32)
    return jnp.concatenate([x[:E], half, x[E:], half]).reshape(E2P // 128, 128)


def kernel(triples, node_embeddings, weights1, bases1, weights2, bases2,
           bias1, bias2):
    s0 = triples[:, 0]
    r0 = triples[:, 1]
    o0 = triples[:, 2]

    col_idx = jnp.concatenate([o0 * R + r0, s0 * R + (r0 + NREL)])
    obj_idx = jnp.concatenate([o0, s0])
    ver_idx = jnp.concatenate([r0 * N + s0, r0 * N + o0])

    pad_ver = HALF + (jnp.arange(PADC, dtype=jnp.int32) % 8)
    pv = jnp.concatenate([ver_idx[:E], pad_ver, ver_idx[E:], pad_ver])
    ver2d = pv.reshape(E2P // 128, 128)
    col2d = _pad_to_rows(col_idx, 0)
    obj2d = _pad_to_rows(obj_idx, 0)

    w1p, w2p = pl.pallas_call(
        _wts_body,
        out_shape=[jax.ShapeDtypeStruct((R, D * H), jnp.float32),
                   jax.ShapeDtypeStruct((R, H * C), jnp.float32)],
    )(weights1, bases1.reshape(NB, D * H), weights2, bases2.reshape(NB, H * C))
    w1flat = w1p.reshape(R, D, H).transpose(1, 0, 2).reshape(D, R * H)
    w2stack = w2p.reshape(R * H, C)

    h = pl.pallas_call(
        _h_body,
        grid=(N // _B1,),
        in_specs=[pl.BlockSpec((_B1, D), lambda i: (i, 0)),
                  pl.BlockSpec((D, R * H), lambda i: (0, 0))],
        out_specs=pl.BlockSpec((_B1, R * H), lambda i: (i, 0)),
        out_shape=jax.ShapeDtypeStruct((N, R * H), jnp.float32),
    )(node_embeddings, w1flat)
    htab = h.reshape(N * R, H)
    hself = h[:, 16 * H:17 * H]

    g, cntw = _make_edge_scatter(N * R, True)(col2d, ver2d, htab)
    cnt = jnp.concatenate([cntw[:HALF], cntw[CSPW:CSPW + HALF]])

    h1 = pl.pallas_call(
        _l1_body,
        grid=(N // _B2,),
        in_specs=[pl.BlockSpec((16, _B2, H), lambda i: (0, i, 0)),
                  pl.BlockSpec((16, _B2, 1), lambda i: (0, i, 0)),
                  pl.BlockSpec((_B2, H), lambda i: (i, 0)),
                  pl.BlockSpec((1, H), lambda i: (0, 0))],
        out_specs=pl.BlockSpec((_B2, H), lambda i: (i, 0)),
        out_shape=jax.ShapeDtypeStruct((N, H), jnp.float32),
    )(g.reshape(16, N, H), cnt.reshape(16, N, 1), hself,
      bias1.reshape(1, H))

    u = _make_edge_scatter(N, False)(obj2d, ver2d, h1)
    if isinstance(u, (list, tuple)):
        u = u[0]

    out = pl.pallas_call(
        _l2_body,
        grid=(N // _B3,),
        in_specs=[pl.BlockSpec((16, _B3, H), lambda i: (0, i, 0)),
                  pl.BlockSpec((16, _B3, 1), lambda i: (0, i, 0)),
                  pl.BlockSpec((_B3, H), lambda i: (i, 0)),
                  pl.BlockSpec((R * H, C), lambda i: (0, 0)),
                  pl.BlockSpec((1, C), lambda i: (0, 0))],
        out_specs=pl.BlockSpec((_B3, C), lambda i: (i, 0)),
        out_shape=jax.ShapeDtypeStruct((N, C), jnp.float32),
    )(u.reshape(16, N, H), cnt.reshape(16, N, 1), h1, w2stack,
      bias2.reshape(1, C))
    return out


# speedup vs baseline: 46.7340x; 1.1098x over previous
import functools

import jax
import jax.numpy as jnp
from jax import lax
from jax.experimental import pallas as pl
from jax.experimental.pallas import tpu as pltpu
from jax.experimental.pallas import tpu_sc as plsc

N = 10000
NREL = 8
R = 2 * NREL + 1
D = 128
H = 16
C = 50
NB = 16

E = 320000
BINS = 16 * N
HALF = 8 * N

NCORES = 2
NSUB = 16
GRP = 1024
CH = GRP // 128
GROUPS = 20
EPT = GRP * GROUPS
EP = EPT * NSUB
EROWS = EP // 128
RPT = HALF // NSUB
CSPW = 16 * 5120

_B1 = 2000
_B2 = 1000
_B3 = 1000


def _wts_body(w1_ref, b1_ref, w2_ref, b2_ref, o1_ref, o2_ref):
    o1_ref[...] = jnp.dot(w1_ref[...], b1_ref[...],
                          preferred_element_type=jnp.float32)
    o2_ref[...] = jnp.dot(w2_ref[...], b2_ref[...],
                          preferred_element_type=jnp.float32)


def _h_body(emb_ref, w_ref, o_ref):
    o_ref[...] = jnp.dot(emb_ref[...], w_ref[...],
                         preferred_element_type=jnp.float32)


def _l1_body(g_ref, c_ref, hs_ref, b1_ref, o_ref):
    acc = hs_ref[...] + b1_ref[...]
    for r in range(16):
        cr = c_ref[r]
        cinv = jnp.where(cr > 0.0, 1.0 / cr, 0.0)
        acc = acc + g_ref[r] * cinv
    o_ref[...] = jnp.maximum(acc, 0.0)


def _l2_body(u_ref, c_ref, h1_ref, w2_ref, b2_ref, o_ref):
    parts = []
    for r in range(16):
        cr = c_ref[r]
        cinv = jnp.where(cr > 0.0, 1.0 / cr, 0.0)
        parts.append(u_ref[r] * cinv)
    parts.append(h1_ref[...])
    v = jnp.concatenate(parts, axis=1)
    o_ref[...] = (jnp.dot(v, w2_ref[...], preferred_element_type=jnp.float32)
                  + b2_ref[...])


@functools.lru_cache(maxsize=None)
def _make_edge_scatter(pass_id: int):
    with_counts = pass_id == 1
    mesh = plsc.VectorSubcoreMesh(core_axis_name="c", subcore_axis_name="s")

    out_type = [jax.ShapeDtypeStruct((BINS, H), jnp.float32)]
    if with_counts:
        out_type.append(jax.ShapeDtypeStruct((BINS,), jnp.float32))

    scratch = []
    for _ in range(2):
        scratch += [
            pltpu.VMEM((CH, 128), jnp.int32),
            pltpu.VMEM((CH, 128), jnp.int32),
            pltpu.VMEM((CH, 128), jnp.int32),
            pltpu.VMEM((CH, 128), jnp.int32),
            pltpu.VMEM((CH, 128), jnp.int32),
            pltpu.VMEM((GRP, H), jnp.float32),
            pltpu.SemaphoreType.DMA,
            pltpu.SemaphoreType.DMA,
        ]
    scratch += [
        pltpu.VMEM_SHARED((HALF + 8, H), jnp.float32),
    ]
    if with_counts:
        scratch += [
            pltpu.VMEM((GRP,), jnp.float32),
            pltpu.VMEM((1024,), jnp.float32),
            pltpu.VMEM_SHARED((CSPW,), jnp.float32),
        ]

    @functools.partial(
        pl.kernel, out_type=out_type, mesh=mesh, scratch_types=scratch,
        compiler_params=pltpu.CompilerParams(use_tc_tiling_on_sc=False))
    def sc_kernel(s_hbm, r_hbm, o_hbm, table_hbm, *rest):
        if with_counts:
            g_hbm, cnt_hbm = rest[0], rest[1]
            bufs = (rest[2:10], rest[10:18])
            gsp, ones, zbuf, csp = rest[18:]
        else:
            g_hbm = rest[0]
            bufs = (rest[1:9], rest[9:17])
            gsp = rest[17]
        rows = bufs[0][5]
        c = lax.axis_index("c")
        t = lax.axis_index("s")
        is_inv = c == 1
        trpt = pl.multiple_of(t * RPT, 8)

        def zrow(i, carry):
            rows[i, :] = jnp.zeros((16,), jnp.float32)
            return carry
        lax.fori_loop(0, GRP, zrow, 0)
        for k in range(5):
            sz = 1024 if k < 4 else RPT - 4 * 1024
            pltpu.sync_copy(rows.at[pl.ds(0, sz)],
                            gsp.at[pl.ds(trpt + k * 1024, sz)])

        @pl.when(t == 0)
        def _zero_pad():
            pltpu.sync_copy(rows.at[pl.ds(0, 8)], gsp.at[pl.ds(HALF, 8)])

        if with_counts:
            def fones(i, carry):
                ones[pl.ds(i * 16, 16)] = jnp.full((16,), 1.0, jnp.float32)
                return carry
            lax.fori_loop(0, GRP // 16, fones, 0)

            def zb(i, carry):
                zbuf[pl.ds(i * 16, 16)] = jnp.zeros((16,), jnp.float32)
                return carry
            lax.fori_loop(0, 64, zb, 0)
            tcw = pl.multiple_of(t * 5120, 128)
            for k in range(5):
                pltpu.sync_copy(zbuf, csp.at[pl.ds(tcw + k * 1024, 1024)])

        plsc.subcore_barrier()

        table = table_hbm

        def load_compute(g, buf):
            sbuf, rbuf, obuf, gibuf, verbuf = buf[0], buf[1], buf[2], buf[3], buf[4]
            base = pl.multiple_of(t * (EPT // 128) + g * CH, 8)
            pltpu.sync_copy(s_hbm.at[pl.ds(base, CH)], sbuf)
            pltpu.sync_copy(r_hbm.at[pl.ds(base, CH)], rbuf)
            pltpu.sync_copy(o_hbm.at[pl.ds(base, CH)], obuf)
            ebase = t * EPT + g * GRP

            def crow(j, carry2):
                for k in range(8):
                    sl = pl.ds(k * 16, 16)
                    s = sbuf[j, sl]
                    r = rbuf[j, sl]
                    o = obuf[j, sl]
                    lane = lax.iota(jnp.int32, 16)
                    e = ebase + j * 128 + k * 16 + lane
                    a = jnp.where(is_inv, s, o)
                    b = jnp.where(is_inv, o, s)
                    if pass_id == 1:
                        gi = a * R + r + jnp.where(is_inv, NREL, 0)
                    else:
                        gi = a
                    ver = r * N + b
                    pad = e >= E
                    gi = jnp.where(pad, e & 127, gi)
                    ver = jnp.where(pad, HALF + (lane & 7), ver)
                    gibuf[j, sl] = gi
                    verbuf[j, sl] = ver
                return carry2
            lax.fori_loop(0, CH, crow, 0)

        def fire_gathers(buf):
            gibuf, rws, gsem = buf[3], buf[5], buf[6]
            return [pltpu.async_copy(table.at[gibuf.at[j]],
                                     rws.at[pl.ds(j * 128, 128)], gsem)
                    for j in range(CH)]

        def fire_scatters(buf):
            verbuf, rws, ssem = buf[4], buf[5], buf[7]
            sc = [pltpu.async_copy(rws.at[pl.ds(j * 128, 128)],
                                   gsp.at[verbuf.at[j]], ssem, add=True)
                  for j in range(CH)]
            if with_counts:
                sc += [pltpu.async_copy(ones.at[pl.ds(j * 128, 128)],
                                        csp.at[verbuf.at[j]], ssem, add=True)
                       for j in range(CH)]
            return sc

        A, B = bufs
        gsemA, ssemA = A[6], A[7]
        gsemB, ssemB = B[6], B[7]

        def drain_gathers(buf):
            pltpu.make_async_copy(g_hbm.at[pl.ds(0, GRP)], buf[5],
                                  buf[6]).wait()

        def drain_scatters(buf):
            pltpu.make_async_copy(g_hbm.at[pl.ds(0, GRP)], buf[5],
                                  buf[7]).wait()
            if with_counts:
                pltpu.make_async_copy(cnt_hbm.at[pl.ds(0, GRP)], ones,
                                      buf[7]).wait()

        load_compute(0, A)
        fire_gathers(A)
        npair = GROUPS // 2

        def pair(i, carry):
            g = 2 * i

            @pl.when(i > 0)
            def _drain_b():
                drain_scatters(B)
            load_compute(g + 1, B)
            fire_gathers(B)
            drain_gathers(A)
            fire_scatters(A)

            @pl.when(i < npair - 1)
            def _next_a():
                drain_scatters(A)
                load_compute(g + 2, A)
                fire_gathers(A)
            drain_gathers(B)
            fire_scatters(B)
            return carry

        lax.fori_loop(0, npair, pair, 0)
        drain_scatters(A)
        drain_scatters(B)

        plsc.subcore_barrier()

        ob = pl.multiple_of(c * HALF + trpt, 8)
        for k in range(5):
            sz = 1024 if k < 4 else RPT - 4 * 1024
            pltpu.sync_copy(gsp.at[pl.ds(trpt + k * 1024, sz)],
                            rows.at[pl.ds(0, sz)])
            pltpu.sync_copy(rows.at[pl.ds(0, sz)],
                            g_hbm.at[pl.ds(ob + k * 1024, sz)])
        if with_counts:
            tcw = pl.multiple_of(t * 5120, 128)
            cb = pl.multiple_of(c * HALF, 128)

            def wb_cnt(k, sz):
                pltpu.sync_copy(csp.at[pl.ds(tcw + k * 1024, sz)],
                                zbuf.at[pl.ds(0, sz)])
                pltpu.sync_copy(zbuf.at[pl.ds(0, sz)],
                                cnt_hbm.at[pl.ds(cb + tcw + k * 1024, sz)])

            @pl.when(t < 15)
            def _wb_full():
                for k in range(5):
                    wb_cnt(k, 1024)

            @pl.when(t == 15)
            def _wb_clip():
                for k in range(3):
                    wb_cnt(k, 1024)
                wb_cnt(3, 128)

    return sc_kernel


def kernel(triples, node_embeddings, weights1, bases1, weights2, bases2,
           bias1, bias2):
    tp = jnp.pad(triples, ((0, EP - E), (0, 0))).T
    s2d = tp[0].reshape(EROWS, 128)
    r2d = tp[1].reshape(EROWS, 128)
    o2d = tp[2].reshape(EROWS, 128)

    w1p, w2p = pl.pallas_call(
        _wts_body,
        out_shape=[jax.ShapeDtypeStruct((R, D * H), jnp.float32),
                   jax.ShapeDtypeStruct((R, H * C), jnp.float32)],
    )(weights1, bases1.reshape(NB, D * H), weights2, bases2.reshape(NB, H * C))
    w1flat = w1p.reshape(R, D, H).transpose(1, 0, 2).reshape(D, R * H)
    w2stack = w2p.reshape(R * H, C)

    h = pl.pallas_call(
        _h_body,
        grid=(N // _B1,),
        in_specs=[pl.BlockSpec((_B1, D), lambda i: (i, 0)),
                  pl.BlockSpec((D, R * H), lambda i: (0, 0))],
        out_specs=pl.BlockSpec((_B1, R * H), lambda i: (i, 0)),
        out_shape=jax.ShapeDtypeStruct((N, R * H), jnp.float32),
    )(node_embeddings, w1flat)
    htab = h.reshape(N * R, H)
    hself = h[:, 16 * H:17 * H]

    g, cnt = _make_edge_scatter(1)(s2d, r2d, o2d, htab)

    h1 = pl.pallas_call(
        _l1_body,
        grid=(N // _B2,),
        in_specs=[pl.BlockSpec((16, _B2, H), lambda i: (0, i, 0)),
                  pl.BlockSpec((16, _B2, 1), lambda i: (0, i, 0)),
                  pl.BlockSpec((_B2, H), lambda i: (i, 0)),
                  pl.BlockSpec((1, H), lambda i: (0, 0))],
        out_specs=pl.BlockSpec((_B2, H), lambda i: (i, 0)),
        out_shape=jax.ShapeDtypeStruct((N, H), jnp.float32),
    )(g.reshape(16, N, H), cnt.reshape(16, N, 1), hself,
      bias1.reshape(1, H))

    u = _make_edge_scatter(2)(s2d, r2d, o2d, h1)
    if isinstance(u, (list, tuple)):
        u = u[0]

    out = pl.pallas_call(
        _l2_body,
        grid=(N // _B3,),
        in_specs=[pl.BlockSpec((16, _B3, H), lambda i: (0, i, 0)),
                  pl.BlockSpec((16, _B3, 1), lambda i: (0, i, 0)),
                  pl.BlockSpec((_B3, H), lambda i: (i, 0)),
                  pl.BlockSpec((R * H, C), lambda i: (0, 0)),
                  pl.BlockSpec((1, C), lambda i: (0, 0))],
        out_specs=pl.BlockSpec((_B3, C), lambda i: (i, 0)),
        out_shape=jax.ShapeDtypeStruct((N, C), jnp.float32),
    )(u.reshape(16, N, H), cnt.reshape(16, N, 1), h1, w2stack,
      bias2.reshape(1, C))
    return out


# speedup vs baseline: 93.2268x; 1.9948x over previous
import functools

import jax
import jax.numpy as jnp
from jax import lax
from jax.experimental import pallas as pl
from jax.experimental.pallas import tpu as pltpu
from jax.experimental.pallas import tpu_sc as plsc

N = 10000
NREL = 8
R = 2 * NREL + 1
D = 128
H = 16
C = 50
NB = 16

E = 320000
NP = 10240
BINS = 16 * NP
HALF = 8 * NP

NCORES = 2
NSUB = 16
GRP = 1024
CH = GRP // 128
GROUPS = 20
EPT = GRP * GROUPS
EP = EPT * NSUB
EROWS = EP // 128
RPT = HALF // NSUB

_B1 = 2000
_B2 = 1000
_B3 = 1000


def _wts_body(w1_ref, b1_ref, w2_ref, b2_ref, o1_ref, o2_ref):
    o1_ref[...] = jnp.dot(w1_ref[...], b1_ref[...],
                          preferred_element_type=jnp.float32)
    o2_ref[...] = jnp.dot(w2_ref[...], b2_ref[...],
                          preferred_element_type=jnp.float32)


def _h_body(emb_ref, w_ref, o_ref):
    o_ref[...] = jnp.dot(emb_ref[...], w_ref[...],
                         preferred_element_type=jnp.float32)


def _cinv_body(c_ref, e_ref, o_ref):
    c = c_ref[...]
    cinv = jnp.where(c > 0.0, 1.0 / c, 0.0)
    o_ref[...] = jnp.dot(cinv, e_ref[...], preferred_element_type=jnp.float32)


def _l1_body(g_ref, ce_ref, hs_ref, b1_ref, o_ref):
    acc = hs_ref[...] + b1_ref[...]
    for r in range(16):
        acc = acc + g_ref[r] * ce_ref[r]
    o_ref[...] = jnp.maximum(acc, 0.0)


def _l2_body(u_ref, ce_ref, h1_ref, w2_ref, b2_ref, o_ref):
    acc = jnp.dot(h1_ref[...], w2_ref[16],
                  preferred_element_type=jnp.float32) + b2_ref[...]
    for r in range(16):
        acc = acc + jnp.dot(u_ref[r] * ce_ref[r], w2_ref[r],
                            preferred_element_type=jnp.float32)
    o_ref[...] = acc


@functools.lru_cache(maxsize=None)
def _make_edge_scatter(pass_id: int):
    with_counts = pass_id == 1
    mesh = plsc.VectorSubcoreMesh(core_axis_name="c", subcore_axis_name="s")

    out_type = [jax.ShapeDtypeStruct((BINS, H), jnp.float32)]
    if with_counts:
        out_type.append(jax.ShapeDtypeStruct((BINS,), jnp.float32))

    scratch = [
        pltpu.VMEM((CH, 128), jnp.int32),
        pltpu.VMEM((CH, 128), jnp.int32),
        pltpu.VMEM((CH, 128), jnp.int32),
    ]
    for _ in range(2):
        scratch += [
            pltpu.VMEM((CH, 128), jnp.int32),
            pltpu.VMEM((CH, 128), jnp.int32),
            pltpu.VMEM((GRP, H), jnp.float32),
            pltpu.SemaphoreType.DMA,
            pltpu.SemaphoreType.DMA,
        ]
    scratch += [
        pltpu.VMEM_SHARED((HALF + 8, H), jnp.float32),
    ]
    if with_counts:
        scratch += [
            pltpu.VMEM((GRP,), jnp.float32),
            pltpu.VMEM((1024,), jnp.float32),
            pltpu.VMEM_SHARED((HALF + 8,), jnp.float32),
        ]

    @functools.partial(
        pl.kernel, out_type=out_type, mesh=mesh, scratch_types=scratch,
        compiler_params=pltpu.CompilerParams(use_tc_tiling_on_sc=False))
    def sc_kernel(s_hbm, r_hbm, o_hbm, table_hbm, *rest):
        if with_counts:
            g_hbm, cnt_hbm = rest[0], rest[1]
            sbuf, rbuf, obuf = rest[2:5]
            bufs = (rest[5:10], rest[10:15])
            gsp, ones, zbuf, csp = rest[15:]
        else:
            g_hbm = rest[0]
            sbuf, rbuf, obuf = rest[1:4]
            bufs = (rest[4:9], rest[9:14])
            gsp = rest[14]
        rows = bufs[0][2]
        c = lax.axis_index("c")
        t = lax.axis_index("s")
        is_inv = c == 1
        trpt = pl.multiple_of(t * RPT, 8)

        def zrow(i, carry):
            rows[i, :] = jnp.zeros((16,), jnp.float32)
            return carry
        lax.fori_loop(0, GRP, zrow, 0)
        for k in range(5):
            pltpu.sync_copy(rows.at[pl.ds(0, 1024)],
                            gsp.at[pl.ds(trpt + k * 1024, 1024)])

        @pl.when(t == 0)
        def _zero_pad():
            pltpu.sync_copy(rows.at[pl.ds(0, 8)], gsp.at[pl.ds(HALF, 8)])

        if with_counts:
            def fones(i, carry):
                ones[pl.ds(i * 16, 16)] = jnp.full((16,), 1.0, jnp.float32)
                return carry
            lax.fori_loop(0, GRP // 16, fones, 0)

            def zb(i, carry):
                zbuf[pl.ds(i * 16, 16)] = jnp.zeros((16,), jnp.float32)
                return carry
            lax.fori_loop(0, 64, zb, 0)
            for k in range(5):
                pltpu.sync_copy(zbuf, csp.at[pl.ds(trpt + k * 1024, 1024)])

        plsc.subcore_barrier()

        table = table_hbm

        def load_compute(g, buf):
            gibuf, verbuf = buf[0], buf[1]
            base = pl.multiple_of(t * (EPT // 128) + g * CH, 8)
            pltpu.sync_copy(s_hbm.at[pl.ds(base, CH)], sbuf)
            pltpu.sync_copy(r_hbm.at[pl.ds(base, CH)], rbuf)
            pltpu.sync_copy(o_hbm.at[pl.ds(base, CH)], obuf)
            ebase = t * EPT + g * GRP

            def crow(j, carry2):
                for k in range(8):
                    sl = pl.ds(k * 16, 16)
                    s = sbuf[j, sl]
                    r = rbuf[j, sl]
                    o = obuf[j, sl]
                    lane = lax.iota(jnp.int32, 16)
                    e = ebase + j * 128 + k * 16 + lane
                    a = jnp.where(is_inv, s, o)
                    b = jnp.where(is_inv, o, s)
                    if pass_id == 1:
                        gi = a * R + r + jnp.where(is_inv, NREL, 0)
                    else:
                        gi = a
                    ver = r * NP + b
                    pad = e >= E
                    gi = jnp.where(pad, e & 127, gi)
                    ver = jnp.where(pad, HALF + (lane & 7), ver)
                    gibuf[j, sl] = gi
                    verbuf[j, sl] = ver
                return carry2
            lax.fori_loop(0, CH, crow, 0)

        def fire_gathers(buf):
            gibuf, rws, gsem = buf[0], buf[2], buf[3]
            return [pltpu.async_copy(table.at[gibuf.at[j]],
                                     rws.at[pl.ds(j * 128, 128)], gsem)
                    for j in range(CH)]

        def fire_scatters(buf):
            verbuf, rws, ssem = buf[1], buf[2], buf[4]
            sc = [pltpu.async_copy(rws.at[pl.ds(j * 128, 128)],
                                   gsp.at[verbuf.at[j]], ssem, add=True)
                  for j in range(CH)]
            if with_counts:
                sc += [pltpu.async_copy(ones.at[pl.ds(j * 128, 128)],
                                        csp.at[verbuf.at[j]], ssem, add=True)
                       for j in range(CH)]
            return sc

        A, B = bufs

        def drain_gathers(buf):
            pltpu.make_async_copy(g_hbm.at[pl.ds(0, GRP)], buf[2],
                                  buf[3]).wait()

        def drain_scatters(buf):
            pltpu.make_async_copy(g_hbm.at[pl.ds(0, GRP)], buf[2],
                                  buf[4]).wait()
            if with_counts:
                pltpu.make_async_copy(cnt_hbm.at[pl.ds(0, GRP)], ones,
                                      buf[4]).wait()

        load_compute(0, A)
        fire_gathers(A)
        npair = GROUPS // 2

        def pair(i, carry):
            g = 2 * i

            @pl.when(i > 0)
            def _drain_b():
                drain_scatters(B)
            load_compute(g + 1, B)
            fire_gathers(B)
            drain_gathers(A)
            fire_scatters(A)

            @pl.when(i < npair - 1)
            def _next_a():
                drain_scatters(A)
                load_compute(g + 2, A)
                fire_gathers(A)
            drain_gathers(B)
            fire_scatters(B)
            return carry

        lax.fori_loop(0, npair, pair, 0)
        drain_scatters(A)
        drain_scatters(B)

        plsc.subcore_barrier()

        ob = pl.multiple_of(c * HALF + trpt, 8)
        for k in range(5):
            pltpu.sync_copy(gsp.at[pl.ds(trpt + k * 1024, 1024)],
                            rows.at[pl.ds(0, 1024)])
            pltpu.sync_copy(rows.at[pl.ds(0, 1024)],
                            g_hbm.at[pl.ds(ob + k * 1024, 1024)])
        if with_counts:
            for k in range(5):
                pltpu.sync_copy(csp.at[pl.ds(trpt + k * 1024, 1024)], zbuf)
                pltpu.sync_copy(zbuf, cnt_hbm.at[pl.ds(ob + k * 1024, 1024)])

    return sc_kernel


def kernel(triples, node_embeddings, weights1, bases1, weights2, bases2,
           bias1, bias2):
    tp = jnp.pad(triples, ((0, EP - E), (0, 0))).T
    s2d = tp[0].reshape(EROWS, 128)
    r2d = tp[1].reshape(EROWS, 128)
    o2d = tp[2].reshape(EROWS, 128)

    w1p, w2p = pl.pallas_call(
        _wts_body,
        out_shape=[jax.ShapeDtypeStruct((R, D * H), jnp.float32),
                   jax.ShapeDtypeStruct((R, H * C), jnp.float32)],
    )(weights1, bases1.reshape(NB, D * H), weights2, bases2.reshape(NB, H * C))
    w1flat = w1p.reshape(R, D, H).transpose(1, 0, 2).reshape(D, R * H)

    h = pl.pallas_call(
        _h_body,
        grid=(N // _B1,),
        in_specs=[pl.BlockSpec((_B1, D), lambda i: (i, 0)),
                  pl.BlockSpec((D, R * H), lambda i: (0, 0))],
        out_specs=pl.BlockSpec((_B1, R * H), lambda i: (i, 0)),
        out_shape=jax.ShapeDtypeStruct((N, R * H), jnp.float32),
    )(node_embeddings, w1flat)
    htab = h.reshape(N * R, H)
    hself = h[:, 16 * H:17 * H]

    g, cnt = _make_edge_scatter(1)(s2d, r2d, o2d, htab)

    f32 = jnp.float32
    exp128 = jnp.kron(jnp.eye(128, dtype=f32), jnp.ones((1, H), f32))
    w2big = jax.vmap(
        lambda m: jnp.kron(jnp.eye(8, dtype=f32), m))(w2p.reshape(R, H, C))
    b1t = jnp.tile(bias1, 8).reshape(1, 8 * H)
    b2t = jnp.tile(bias2, 8).reshape(1, 8 * C)

    NR = BINS // 128
    BR = NR // 5

    cinvexp = pl.pallas_call(
        _cinv_body,
        grid=(5,),
        in_specs=[pl.BlockSpec((BR, 128), lambda i: (i, 0)),
                  pl.BlockSpec((128, 128 * H), lambda i: (0, 0))],
        out_specs=pl.BlockSpec((BR, 128 * H), lambda i: (i, 0)),
        out_shape=jax.ShapeDtypeStruct((NR, 128 * H), f32),
    )(cnt.reshape(NR, 128), exp128)

    g3 = g.reshape(16, NR, 128)
    ce3 = cinvexp.reshape(16, NR, 128)

    h1 = pl.pallas_call(
        _l1_body,
        grid=(5,),
        in_specs=[pl.BlockSpec((16, BR, 128), lambda i: (0, i, 0)),
                  pl.BlockSpec((16, BR, 128), lambda i: (0, i, 0)),
                  pl.BlockSpec((BR, 128), lambda i: (i, 0)),
                  pl.BlockSpec((1, 128), lambda i: (0, 0))],
        out_specs=pl.BlockSpec((BR, 128), lambda i: (i, 0)),
        out_shape=jax.ShapeDtypeStruct((NR, 128), f32),
    )(g3, ce3, jnp.pad(hself, ((0, NP - N), (0, 0))).reshape(NR, 128), b1t)

    u = _make_edge_scatter(2)(s2d, r2d, o2d, h1.reshape(NP, H))
    if isinstance(u, (list, tuple)):
        u = u[0]

    out = pl.pallas_call(
        _l2_body,
        grid=(5,),
        in_specs=[pl.BlockSpec((16, BR, 128), lambda i: (0, i, 0)),
                  pl.BlockSpec((16, BR, 128), lambda i: (0, i, 0)),
                  pl.BlockSpec((BR, 128), lambda i: (i, 0)),
                  pl.BlockSpec((R, 128, 8 * C), lambda i: (0, 0, 0)),
                  pl.BlockSpec((1, 8 * C), lambda i: (0, 0))],
        out_specs=pl.BlockSpec((BR, 8 * C), lambda i: (i, 0)),
        out_shape=jax.ShapeDtypeStruct((NR, 8 * C), f32),
    )(u.reshape(16, NR, 128), ce3, h1, w2big, b2t)
    return out.reshape(NP, C)[:N]


# speedup vs baseline: 108.0193x; 1.1587x over previous
import functools

import jax
import jax.numpy as jnp
from jax import lax
from jax.experimental import pallas as pl
from jax.experimental.pallas import tpu as pltpu
from jax.experimental.pallas import tpu_sc as plsc

N = 10000
NREL = 8
R = 2 * NREL + 1
D = 128
H = 16
C = 50
NB = 16

E = 320000
NP = 10240
BINS = 16 * NP
HALF = 8 * NP

NCORES = 2
NSUB = 16
GRP = 1024
CH = GRP // 128
GROUPS = 20
EPT = GRP * GROUPS
EP = EPT * NSUB
EROWS = EP // 128
RPT = HALF // NSUB

_B1 = 2000
_B2 = 1000
_B3 = 1000


def _wts_body(w1_ref, b1_ref, w2_ref, b2_ref, o1_ref, o2_ref):
    o1_ref[...] = jnp.dot(w1_ref[...], b1_ref[...],
                          preferred_element_type=jnp.float32)
    o2_ref[...] = jnp.dot(w2_ref[...], b2_ref[...],
                          preferred_element_type=jnp.float32)


def _h_body(emb_ref, w_ref, o_ref):
    o_ref[...] = jnp.dot(emb_ref[...], w_ref[...],
                         preferred_element_type=jnp.float32)


def _cinv_body(c_ref, e_ref, o_ref):
    c = c_ref[...]
    cinv = jnp.where(c > 0.0, 1.0 / c, 0.0)
    o_ref[...] = jnp.dot(cinv, e_ref[...], preferred_element_type=jnp.float32)


def _l1_body(g_ref, ce_ref, hs_ref, b1_ref, o_ref):
    acc = hs_ref[...] + b1_ref[...]
    for r in range(16):
        acc = acc + g_ref[r] * ce_ref[r]
    o_ref[...] = jnp.maximum(acc, 0.0)


def _l2_body(u_ref, ce_ref, h1_ref, w2_ref, b2_ref, o_ref):
    acc = jnp.dot(h1_ref[...], w2_ref[16],
                  preferred_element_type=jnp.float32) + b2_ref[...]
    for r in range(16):
        acc = acc + jnp.dot(u_ref[r] * ce_ref[r], w2_ref[r],
                            preferred_element_type=jnp.float32)
    o_ref[...] = acc


@functools.lru_cache(maxsize=None)
def _make_edge_scatter(pass_id: int):
    with_counts = pass_id == 1
    mesh = plsc.VectorSubcoreMesh(core_axis_name="c", subcore_axis_name="s")

    out_type = [jax.ShapeDtypeStruct((BINS, H), jnp.float32)]
    if with_counts:
        out_type.append(jax.ShapeDtypeStruct((BINS,), jnp.float32))

    scratch = [
        pltpu.VMEM((3 * CH, 128), jnp.int32),
    ]
    for _ in range(2):
        scratch += [
            pltpu.VMEM((CH, 128), jnp.int32),
            pltpu.VMEM((CH, 128), jnp.int32),
            pltpu.VMEM((GRP, H), jnp.float32),
            pltpu.SemaphoreType.DMA,
            pltpu.SemaphoreType.DMA,
        ]
    scratch += [
        pltpu.VMEM_SHARED((HALF + 8, H), jnp.float32),
    ]
    if with_counts:
        scratch += [
            pltpu.VMEM((GRP,), jnp.float32),
            pltpu.VMEM((1024,), jnp.float32),
            pltpu.VMEM_SHARED((HALF + 8,), jnp.float32),
        ]

    @functools.partial(
        pl.kernel, out_type=out_type, mesh=mesh, scratch_types=scratch,
        compiler_params=pltpu.CompilerParams(use_tc_tiling_on_sc=False))
    def sc_kernel(srl_hbm, table_hbm, *rest):
        if with_counts:
            g_hbm, cnt_hbm = rest[0], rest[1]
            srl = rest[2]
            bufs = (rest[3:8], rest[8:13])
            gsp, ones, zbuf, csp = rest[13:]
        else:
            g_hbm = rest[0]
            srl = rest[1]
            bufs = (rest[2:7], rest[7:12])
            gsp = rest[12]
        rows = bufs[0][2]
        c = lax.axis_index("c")
        t = lax.axis_index("s")
        is_inv = c == 1
        trpt = pl.multiple_of(t * RPT, 8)

        def zrow(i, carry):
            rows[i, :] = jnp.zeros((16,), jnp.float32)
            return carry
        lax.fori_loop(0, GRP, zrow, 0)
        for k in range(5):
            pltpu.sync_copy(rows.at[pl.ds(0, 1024)],
                            gsp.at[pl.ds(trpt + k * 1024, 1024)])

        @pl.when(t == 0)
        def _zero_pad():
            pltpu.sync_copy(rows.at[pl.ds(0, 8)], gsp.at[pl.ds(HALF, 8)])

        if with_counts:
            def fones(i, carry):
                ones[pl.ds(i * 16, 16)] = jnp.full((16,), 1.0, jnp.float32)
                return carry
            lax.fori_loop(0, GRP // 16, fones, 0)

            def zb(i, carry):
                zbuf[pl.ds(i * 16, 16)] = jnp.zeros((16,), jnp.float32)
                return carry
            lax.fori_loop(0, 64, zb, 0)
            for k in range(5):
                pltpu.sync_copy(zbuf, csp.at[pl.ds(trpt + k * 1024, 1024)])

        plsc.subcore_barrier()

        table = table_hbm

        def load_compute(g, buf):
            gibuf, verbuf = buf[0], buf[1]
            base = pl.multiple_of(3 * (t * (EPT // 128) + g * CH), 8)
            pltpu.sync_copy(srl_hbm.at[pl.ds(base, 3 * CH)], srl)
            ebase = t * EPT + g * GRP

            def crow(j, carry2):
                for k in range(8):
                    sl = pl.ds(k * 16, 16)
                    s = srl[3 * j, sl]
                    r = srl[3 * j + 1, sl]
                    o = srl[3 * j + 2, sl]
                    lane = lax.iota(jnp.int32, 16)
                    e = ebase + j * 128 + k * 16 + lane
                    a = jnp.where(is_inv, s, o)
                    b = jnp.where(is_inv, o, s)
                    if pass_id == 1:
                        gi = a * R + r + jnp.where(is_inv, NREL, 0)
                    else:
                        gi = a
                    ver = r * NP + b
                    pad = e >= E
                    gi = jnp.where(pad, e & 127, gi)
                    ver = jnp.where(pad, HALF + (lane & 7), ver)
                    gibuf[j, sl] = gi
                    verbuf[j, sl] = ver
                return carry2
            lax.fori_loop(0, CH, crow, 0)

        def fire_gathers(buf):
            gibuf, rws, gsem = buf[0], buf[2], buf[3]
            return [pltpu.async_copy(table.at[gibuf.at[j]],
                                     rws.at[pl.ds(j * 128, 128)], gsem)
                    for j in range(CH)]

        def fire_scatters(buf):
            verbuf, rws, ssem = buf[1], buf[2], buf[4]
            sc = [pltpu.async_copy(rws.at[pl.ds(j * 128, 128)],
                                   gsp.at[verbuf.at[j]], ssem, add=True)
                  for j in range(CH)]
            if with_counts:
                sc += [pltpu.async_copy(ones.at[pl.ds(j * 128, 128)],
                                        csp.at[verbuf.at[j]], ssem, add=True)
                       for j in range(CH)]
            return sc

        A, B = bufs

        def drain_gathers(buf):
            pltpu.make_async_copy(g_hbm.at[pl.ds(0, GRP)], buf[2],
                                  buf[3]).wait()

        def drain_scatters(buf):
            pltpu.make_async_copy(g_hbm.at[pl.ds(0, GRP)], buf[2],
                                  buf[4]).wait()
            if with_counts:
                pltpu.make_async_copy(cnt_hbm.at[pl.ds(0, GRP)], ones,
                                      buf[4]).wait()

        load_compute(0, A)
        fire_gathers(A)
        npair = GROUPS // 2

        def pair(i, carry):
            g = 2 * i

            @pl.when(i > 0)
            def _drain_b():
                drain_scatters(B)
            load_compute(g + 1, B)
            fire_gathers(B)
            drain_gathers(A)
            fire_scatters(A)

            @pl.when(i < npair - 1)
            def _next_a():
                drain_scatters(A)
                load_compute(g + 2, A)
                fire_gathers(A)
            drain_gathers(B)
            fire_scatters(B)
            return carry

        lax.fori_loop(0, npair, pair, 0)
        drain_scatters(A)
        drain_scatters(B)

        plsc.subcore_barrier()

        ob = pl.multiple_of(c * HALF + trpt, 8)
        for k in range(5):
            pltpu.sync_copy(gsp.at[pl.ds(trpt + k * 1024, 1024)],
                            rows.at[pl.ds(0, 1024)])
            pltpu.sync_copy(rows.at[pl.ds(0, 1024)],
                            g_hbm.at[pl.ds(ob + k * 1024, 1024)])
        if with_counts:
            for k in range(5):
                pltpu.sync_copy(csp.at[pl.ds(trpt + k * 1024, 1024)], zbuf)
                pltpu.sync_copy(zbuf, cnt_hbm.at[pl.ds(ob + k * 1024, 1024)])

    return sc_kernel


def kernel(triples, node_embeddings, weights1, bases1, weights2, bases2,
           bias1, bias2):
    tp = jnp.pad(triples, ((0, EP - E), (0, 0))).T
    srl = tp.reshape(3, EROWS, 128).transpose(1, 0, 2).reshape(3 * EROWS, 128)

    w1p, w2p = pl.pallas_call(
        _wts_body,
        out_shape=[jax.ShapeDtypeStruct((R, D * H), jnp.float32),
                   jax.ShapeDtypeStruct((R, H * C), jnp.float32)],
    )(weights1, bases1.reshape(NB, D * H), weights2, bases2.reshape(NB, H * C))
    w1flat = w1p.reshape(R, D, H).transpose(1, 0, 2).reshape(D, R * H)

    h = pl.pallas_call(
        _h_body,
        grid=(N // _B1,),
        in_specs=[pl.BlockSpec((_B1, D), lambda i: (i, 0)),
                  pl.BlockSpec((D, R * H), lambda i: (0, 0))],
        out_specs=pl.BlockSpec((_B1, R * H), lambda i: (i, 0)),
        out_shape=jax.ShapeDtypeStruct((N, R * H), jnp.float32),
    )(node_embeddings, w1flat)
    htab = h.reshape(N * R, H)
    hself = h[:, 16 * H:17 * H]

    g, cnt = _make_edge_scatter(1)(srl, htab)

    f32 = jnp.float32
    exp128 = jnp.kron(jnp.eye(128, dtype=f32), jnp.ones((1, H), f32))
    w2big = jax.vmap(
        lambda m: jnp.kron(jnp.eye(8, dtype=f32), m))(w2p.reshape(R, H, C))
    b1t = jnp.tile(bias1, 8).reshape(1, 8 * H)
    b2t = jnp.tile(bias2, 8).reshape(1, 8 * C)

    NR = BINS // 128
    BR = NR // 5

    cinvexp = pl.pallas_call(
        _cinv_body,
        grid=(5,),
        in_specs=[pl.BlockSpec((BR, 128), lambda i: (i, 0)),
                  pl.BlockSpec((128, 128 * H), lambda i: (0, 0))],
        out_specs=pl.BlockSpec((BR, 128 * H), lambda i: (i, 0)),
        out_shape=jax.ShapeDtypeStruct((NR, 128 * H), f32),
    )(cnt.reshape(NR, 128), exp128)

    g3 = g.reshape(16, NR, 128)
    ce3 = cinvexp.reshape(16, NR, 128)

    h1 = pl.pallas_call(
        _l1_body,
        grid=(5,),
        in_specs=[pl.BlockSpec((16, BR, 128), lambda i: (0, i, 0)),
                  pl.BlockSpec((16, BR, 128), lambda i: (0, i, 0)),
                  pl.BlockSpec((BR, 128), lambda i: (i, 0)),
                  pl.BlockSpec((1, 128), lambda i: (0, 0))],
        out_specs=pl.BlockSpec((BR, 128), lambda i: (i, 0)),
        out_shape=jax.ShapeDtypeStruct((NR, 128), f32),
    )(g3, ce3, jnp.pad(hself, ((0, NP - N), (0, 0))).reshape(NR, 128), b1t)

    u = _make_edge_scatter(2)(srl, h1.reshape(NP, H))
    if isinstance(u, (list, tuple)):
        u = u[0]

    out = pl.pallas_call(
        _l2_body,
        grid=(5,),
        in_specs=[pl.BlockSpec((16, BR, 128), lambda i: (0, i, 0)),
                  pl.BlockSpec((16, BR, 128), lambda i: (0, i, 0)),
                  pl.BlockSpec((BR, 128), lambda i: (i, 0)),
                  pl.BlockSpec((R, 128, 8 * C), lambda i: (0, 0, 0)),
                  pl.BlockSpec((1, 8 * C), lambda i: (0, 0))],
        out_specs=pl.BlockSpec((BR, 8 * C), lambda i: (i, 0)),
        out_shape=jax.ShapeDtypeStruct((NR, 8 * C), f32),
    )(u.reshape(16, NR, 128), ce3, h1, w2big, b2t)
    return out.reshape(NP, C)[:N]
